# speedup vs baseline: 3.0523x; 3.0523x over previous
import jax
import jax.numpy as jnp
from jax import lax
from jax.experimental import pallas as pl
from jax.experimental.pallas import tpu as pltpu

B = 4
K = 1024
DQ = 4
C = 32
HW = 1024

_DOT = dict(precision=lax.Precision.HIGHEST, preferred_element_type=jnp.float32)


def _guided_kernel(f_ref, q_ref, wt_ref, sel_ref, codes_ref, loss_ref):
    wq_list = [wt_ref[q] for q in range(DQ)]
    kiota = lax.broadcasted_iota(jnp.int32, (K, 1), 0)
    csum = jnp.float32(0.0)
    for b in range(B):
        F = f_ref[b]
        Q = q_ref[b]
        G = lax.dot_general(F, F, (((1,), (1,)), ((), ())), **_DOT)
        M = lax.dot_general(Q, F, (((1,), (1,)), ((), ())), **_DOT)
        score = jnp.zeros((K, 1), jnp.float32)
        for q in range(DQ):
            Wq = wq_list[q]
            Y = lax.dot_general(Wq, G, (((1,), (0,)), ((), ())), **_DOT)
            Z = 2.0 * M[q:q + 1, :] - Y
            score = score + jnp.sum(Wq * Z, axis=1, keepdims=True)
        smax = jnp.max(score)
        code_b = jnp.min(jnp.where(score == smax, kiota, K))
        oh = jnp.where(kiota == code_b, 1.0, 0.0).astype(jnp.float32)
        wsel_rows = [
            lax.dot_general(oh, wq_list[q], (((0,), (0,)), ((), ())), **_DOT)
            for q in range(DQ)
        ]
        Wsel = jnp.concatenate(wsel_rows, axis=0)
        sel = lax.dot_general(Wsel, F, (((1,), (0,)), ((), ())), **_DOT)
        sel_ref[b] = sel
        csum = csum + jnp.sum((sel - Q) ** 2)
        codes_ref[b] = code_b
    loss_ref[0] = csum / jnp.float32(B * DQ * HW)


def kernel(features, query, W):
    f3 = features.reshape(B, C, HW)
    q3 = query.reshape(B, DQ, HW)
    wt = jnp.transpose(W, (1, 0, 2))
    sel, codes, loss = pl.pallas_call(
        _guided_kernel,
        out_shape=[
            jax.ShapeDtypeStruct((B, DQ, HW), jnp.float32),
            jax.ShapeDtypeStruct((B,), jnp.int32),
            jax.ShapeDtypeStruct((1,), jnp.float32),
        ],
        out_specs=[
            pl.BlockSpec(memory_space=pltpu.VMEM),
            pl.BlockSpec(memory_space=pltpu.SMEM),
            pl.BlockSpec(memory_space=pltpu.SMEM),
        ],
        in_specs=[
            pl.BlockSpec(memory_space=pltpu.VMEM),
            pl.BlockSpec(memory_space=pltpu.VMEM),
            pl.BlockSpec(memory_space=pltpu.VMEM),
        ],
    )(f3, q3, wt)
    return sel.reshape(B, DQ, 32, 32), codes, loss[0]


# speedup vs baseline: 4.0865x; 1.3388x over previous
import jax
import jax.numpy as jnp
from jax import lax
from jax.experimental import pallas as pl
from jax.experimental.pallas import tpu as pltpu

B = 4
K = 1024
DQ = 4
C = 32
HW = 1024

_DOT = dict(precision=lax.Precision.HIGHEST, preferred_element_type=jnp.float32)


def _guided_kernel(f_ref, q_ref, wf_ref, wt_ref, sel_ref, codes_ref, loss_ref):
    Wf = wf_ref[:]
    kiota = lax.broadcasted_iota(jnp.int32, (K, 1), 0)
    sub = lax.broadcasted_iota(jnp.int32, (DQ * C, DQ * C), 0)
    lane = lax.broadcasted_iota(jnp.int32, (DQ * C, DQ * C), 1)
    blkmask = (sub // C) == (lane // C)
    csum = jnp.float32(0.0)
    for b in range(B):
        F = f_ref[b]
        Q = q_ref[b]
        G = lax.dot_general(F, F, (((1,), (1,)), ((), ())), **_DOT)
        M = lax.dot_general(Q, F, (((1,), (1,)), ((), ())), **_DOT)
        grow = jnp.concatenate([G] * DQ, axis=1)
        gbig = jnp.concatenate([grow] * DQ, axis=0)
        Gd = jnp.where(blkmask, gbig, 0.0)
        Y = lax.dot_general(Wf, Gd, (((1,), (0,)), ((), ())), **_DOT)
        mrow = jnp.concatenate([M[q:q + 1, :] for q in range(DQ)], axis=1)
        score = jnp.sum(Wf * (2.0 * mrow - Y), axis=1, keepdims=True)
        smax = jnp.max(score)
        code_b = jnp.min(jnp.where(score == smax, kiota, K))
        Wsel = jnp.concatenate(
            [wt_ref[q, pl.ds(code_b, 1), :] for q in range(DQ)], axis=0
        )
        sel = lax.dot_general(Wsel, F, (((1,), (0,)), ((), ())), **_DOT)
        sel_ref[b] = sel
        csum = csum + jnp.sum((sel - Q) ** 2)
        codes_ref[b] = code_b
    loss_ref[0] = csum / jnp.float32(B * DQ * HW)


def kernel(features, query, W):
    f3 = features.reshape(B, C, HW)
    q3 = query.reshape(B, DQ, HW)
    wf = W.reshape(K, DQ * C)
    wt = jnp.transpose(W, (1, 0, 2))
    sel, codes, loss = pl.pallas_call(
        _guided_kernel,
        out_shape=[
            jax.ShapeDtypeStruct((B, DQ, HW), jnp.float32),
            jax.ShapeDtypeStruct((B,), jnp.int32),
            jax.ShapeDtypeStruct((1,), jnp.float32),
        ],
        out_specs=[
            pl.BlockSpec(memory_space=pltpu.VMEM),
            pl.BlockSpec(memory_space=pltpu.SMEM),
            pl.BlockSpec(memory_space=pltpu.SMEM),
        ],
        in_specs=[
            pl.BlockSpec(memory_space=pltpu.VMEM),
            pl.BlockSpec(memory_space=pltpu.VMEM),
            pl.BlockSpec(memory_space=pltpu.VMEM),
            pl.BlockSpec(memory_space=pltpu.VMEM),
        ],
    )(f3, q3, wf, wt)
    return sel.reshape(B, DQ, 32, 32), codes, loss[0]


# speedup vs baseline: 6.2919x; 1.5397x over previous
import jax
import jax.numpy as jnp
from jax import lax
from jax.experimental import pallas as pl
from jax.experimental.pallas import tpu as pltpu

B = 4
K = 1024
DQ = 4
C = 32
HW = 1024

_DOT = dict(precision=lax.Precision.HIGHEST, preferred_element_type=jnp.float32)


def _guided_kernel(f_ref, q_ref, wf_ref, sel_ref, codes_ref, loss_ref):
    Wf = wf_ref[:]
    kiota = lax.broadcasted_iota(jnp.int32, (K, 1), 0)
    sub = lax.broadcasted_iota(jnp.int32, (DQ * C, DQ * C), 0)
    lane = lax.broadcasted_iota(jnp.int32, (DQ * C, DQ * C), 1)
    blkmask = (sub // C) == (lane // C)
    csum = jnp.float32(0.0)
    for b in range(B):
        F = f_ref[b].reshape(C, HW)
        Q = q_ref[b].reshape(DQ, HW)
        G = lax.dot_general(F, F, (((1,), (1,)), ((), ())), **_DOT)
        M = lax.dot_general(Q, F, (((1,), (1,)), ((), ())), **_DOT)
        grow = jnp.concatenate([G] * DQ, axis=1)
        gbig = jnp.concatenate([grow] * DQ, axis=0)
        Gd = jnp.where(blkmask, gbig, 0.0)
        Y = lax.dot_general(Wf, Gd, (((1,), (0,)), ((), ())), **_DOT)
        mrow = jnp.concatenate([M[q:q + 1, :] for q in range(DQ)], axis=1)
        score = jnp.sum(Wf * (2.0 * mrow - Y), axis=1, keepdims=True)
        smax = jnp.max(score)
        code_b = jnp.min(jnp.where(score == smax, kiota, K))
        wrow = wf_ref[pl.ds(code_b, 1), :]
        Wsel = jnp.concatenate(
            [wrow[:, q * C:(q + 1) * C] for q in range(DQ)], axis=0
        )
        sel = lax.dot_general(Wsel, F, (((1,), (0,)), ((), ())), **_DOT)
        sel_ref[b] = sel.reshape(DQ, 32, 32)
        csum = csum + jnp.sum((sel - Q) ** 2)
        codes_ref[b] = code_b
    loss_ref[0] = csum / jnp.float32(B * DQ * HW)


def kernel(features, query, W):
    wf = W.reshape(K, DQ * C)
    sel, codes, loss = pl.pallas_call(
        _guided_kernel,
        out_shape=[
            jax.ShapeDtypeStruct((B, DQ, 32, 32), jnp.float32),
            jax.ShapeDtypeStruct((B,), jnp.int32),
            jax.ShapeDtypeStruct((1,), jnp.float32),
        ],
        out_specs=[
            pl.BlockSpec(memory_space=pltpu.VMEM),
            pl.BlockSpec(memory_space=pltpu.SMEM),
            pl.BlockSpec(memory_space=pltpu.SMEM),
        ],
        in_specs=[
            pl.BlockSpec(memory_space=pltpu.VMEM),
            pl.BlockSpec(memory_space=pltpu.VMEM),
            pl.BlockSpec(memory_space=pltpu.VMEM),
        ],
    )(features, query, wf)
    return sel, codes, loss[0]


# speedup vs baseline: 7.4506x; 1.1842x over previous
import jax
import jax.numpy as jnp
from jax import lax
from jax.experimental import pallas as pl
from jax.experimental.pallas import tpu as pltpu

B = 4
K = 1024
DQ = 4
C = 32
HW = 1024

_DOT = dict(precision=lax.Precision.HIGHEST, preferred_element_type=jnp.float32)


def _guided_kernel(f_ref, q_ref, wf_ref, sel_ref, codes_ref, loss_ref):
    Wf = wf_ref[:]
    Fall = f_ref[:].reshape(B * C, HW)
    Qall = q_ref[:].reshape(B * DQ, HW)
    Gall = lax.dot_general(Fall, Fall, (((1,), (1,)), ((), ())), **_DOT)
    Mall = lax.dot_general(Qall, Fall, (((1,), (1,)), ((), ())), **_DOT)

    sub = lax.broadcasted_iota(jnp.int32, (DQ * C, B * DQ * C), 0)
    lane = lax.broadcasted_iota(jnp.int32, (DQ * C, B * DQ * C), 1)
    keep = (sub // C) == ((lane // C) % DQ)
    gcols = []
    for b in range(B):
        Gb = Gall[b * C:(b + 1) * C, b * C:(b + 1) * C]
        grow = jnp.concatenate([Gb] * DQ, axis=1)
        gcols.append(jnp.concatenate([grow] * DQ, axis=0))
    Gbig = jnp.where(keep, jnp.concatenate(gcols, axis=1), 0.0)
    Y = lax.dot_general(Wf, Gbig, (((1,), (0,)), ((), ())), **_DOT)

    mparts = []
    for b in range(B):
        for q in range(DQ):
            mparts.append(Mall[b * DQ + q:b * DQ + q + 1, b * C:(b + 1) * C])
    Mbig = jnp.concatenate(mparts, axis=1)

    Wtile = jnp.concatenate([Wf] * B, axis=1)
    E = Wtile * (2.0 * Mbig - Y)
    score = jnp.sum(E.reshape(K, B, DQ * C), axis=2)

    kiota = lax.broadcasted_iota(jnp.int32, (K, B), 0)
    smax = jnp.max(score, axis=0, keepdims=True)
    codes_row = jnp.min(jnp.where(score == smax, kiota, K), axis=0)

    wrows = []
    zer = jnp.zeros((DQ, DQ * C), jnp.float32)
    for b in range(B):
        code_b = codes_row[b]
        codes_ref[b] = code_b
        wrow = wf_ref[pl.ds(code_b, 1), :]
        Wsel = jnp.concatenate(
            [wrow[:, q * C:(q + 1) * C] for q in range(DQ)], axis=0
        )
        pads = [zer[:, :b * C], Wsel, zer[:, (b + 1) * C:]]
        wrows.append(jnp.concatenate([p for p in pads if p.shape[1]], axis=1))
    Wbig = jnp.concatenate(wrows, axis=0)
    selall = lax.dot_general(Wbig, Fall, (((1,), (0,)), ((), ())), **_DOT)
    sel_ref[:] = selall.reshape(B, DQ, 32, 32)
    loss_ref[0] = jnp.sum((selall - Qall) ** 2) / jnp.float32(B * DQ * HW)


def kernel(features, query, W):
    wf = W.reshape(K, DQ * C)
    sel, codes, loss = pl.pallas_call(
        _guided_kernel,
        out_shape=[
            jax.ShapeDtypeStruct((B, DQ, 32, 32), jnp.float32),
            jax.ShapeDtypeStruct((B,), jnp.int32),
            jax.ShapeDtypeStruct((1,), jnp.float32),
        ],
        out_specs=[
            pl.BlockSpec(memory_space=pltpu.VMEM),
            pl.BlockSpec(memory_space=pltpu.SMEM),
            pl.BlockSpec(memory_space=pltpu.SMEM),
        ],
        in_specs=[
            pl.BlockSpec(memory_space=pltpu.VMEM),
            pl.BlockSpec(memory_space=pltpu.VMEM),
            pl.BlockSpec(memory_space=pltpu.VMEM),
        ],
    )(features, query, wf)
    return sel, codes, loss[0]
